# Initial kernel scaffold; baseline (speedup 1.0000x reference)
import jax
import jax.numpy as jnp
from jax.experimental import pallas as pl


def kernel(logits, e_score_correction_bias):
    raise NotImplementedError("write your pallas kernel here")


# speedup vs baseline: 36.0874x; 36.0874x over previous
import functools

import jax
import jax.numpy as jnp
from jax import lax
from jax.experimental import pallas as pl
from jax.experimental.pallas import tpu as pltpu
from jax.experimental.pallas import tpu_sc as plsc

T = 16384
E = 256
K = 8
NGROUP = 8
GSIZE = E // NGROUP
NC, NS, L = 2, 16, 16
NW = NC * NS
TPW = T // NW
CHUNK = 128
NCHUNK = TPW // CHUNK
NEG_INF = float("-inf")


def _merge_top16(ka, va, kb, vb):
    kr = lax.rev(kb, (0,))
    vr = lax.rev(vb, (0,))
    ge = ka >= kr
    hk = jnp.where(ge, ka, kr)
    hv = jnp.where(ge, va, vr)
    return plsc.sort_key_val(hk, hv, descending=True)


def _routing_body(lf_hbm, bias_hbm, oi_hbm, ov_hbm,
                  lbuf, bias_v, sc_scr, swb_scr, oi_scr, ov_scr):
    wid = lax.axis_index("s") * NC + lax.axis_index("c")
    iota = lax.iota(jnp.int32, L)
    pltpu.sync_copy(bias_hbm, bias_v)
    tok0 = wid * TPW

    @pl.loop(0, NCHUNK)
    def _chunk(ci):
        base_tok = tok0 + ci * CHUNK
        pltpu.sync_copy(lf_hbm.at[pl.ds(base_tok * E, CHUNK * E)], lbuf)

        @pl.loop(0, CHUNK)
        def _tok(t):
            toff = t * E
            for j in range(E // L):
                x = lbuf[pl.ds(toff + j * L, L)]
                s = 1.0 / (1.0 + jnp.exp(-x))
                sc_scr[pl.ds(j * L, L)] = s
                swb_scr[pl.ds(j * L, L)] = s + bias_v[pl.ds(j * L, L)]
            gv = jnp.full((L,), NEG_INF, jnp.float32)
            for g in range(NGROUP):
                a = swb_scr[pl.ds(g * GSIZE, L)]
                b = swb_scr[pl.ds(g * GSIZE + L, L)]
                hi = jnp.maximum(a, b)
                lo = jnp.minimum(a, b)
                m1 = jnp.max(hi)
                lane = plsc.all_reduce_ffs(hi == m1)
                hi2 = jnp.where(iota == lane, lo, hi)
                m2 = jnp.max(hi2)
                gv = jnp.where(iota == g, m1 + m2, gv)
            kept = []
            for r in range(4):
                m = jnp.max(gv)
                lanev = plsc.all_reduce_ffs(gv == m)
                gr = jnp.max(lanev)
                gv = jnp.where(iota == lanev, NEG_INF, gv)
                gbase = gr * GSIZE
                a = swb_scr[pl.ds(gbase, L)]
                b = swb_scr[pl.ds(gbase + L, L)]
                ida = gbase + iota
                idb = ida + L
                ka, va = plsc.sort_key_val(a, ida, descending=True)
                kb, vb = plsc.sort_key_val(b, idb, descending=True)
                kept.append(_merge_top16(ka, va, kb, vb))
            u0 = _merge_top16(*kept[0], *kept[1])
            u1 = _merge_top16(*kept[2], *kept[3])
            fk, fv = _merge_top16(*u0, *u1)
            mask8 = iota < K
            sgath = plsc.load_gather(sc_scr, [fv])
            s8 = jnp.where(mask8, sgath, 0.0)
            denom = jnp.broadcast_to(jnp.sum(s8) + 1e-20, (L,))
            vals = s8 * 2.5 / denom
            keys = jnp.where(mask8, vals, -1.0)
            ok, oi = plsc.sort_key_val(keys, fv, descending=True)
            plsc.store_compressed(ov_scr.at[pl.ds(t * K, L)], ok, mask=mask8)
            plsc.store_compressed(oi_scr.at[pl.ds(t * K, L)], oi, mask=mask8)

        pltpu.sync_copy(ov_scr.at[pl.ds(0, CHUNK * K)],
                        ov_hbm.at[pl.ds(base_tok * K, CHUNK * K)])
        pltpu.sync_copy(oi_scr.at[pl.ds(0, CHUNK * K)],
                        oi_hbm.at[pl.ds(base_tok * K, CHUNK * K)])


_router = functools.partial(
    pl.kernel,
    out_type=(
        jax.ShapeDtypeStruct((T * K,), jnp.int32),
        jax.ShapeDtypeStruct((T * K,), jnp.float32),
    ),
    mesh=plsc.VectorSubcoreMesh(
        core_axis_name="c", subcore_axis_name="s", num_cores=NC, num_subcores=NS
    ),
    compiler_params=pltpu.CompilerParams(needs_layout_passes=False),
    scratch_types=[
        pltpu.VMEM((CHUNK * E,), jnp.float32),
        pltpu.VMEM((E,), jnp.float32),
        pltpu.VMEM((E,), jnp.float32),
        pltpu.VMEM((E,), jnp.float32),
        pltpu.VMEM((CHUNK * K + K,), jnp.int32),
        pltpu.VMEM((CHUNK * K + K,), jnp.float32),
    ],
)(_routing_body)


def kernel(logits, e_score_correction_bias):
    oi, ov = _router(logits.reshape(-1), e_score_correction_bias)
    return oi.reshape(T, K), ov.reshape(T, K)


# speedup vs baseline: 36.9944x; 1.0251x over previous
import functools

import jax
import jax.numpy as jnp
from jax import lax
from jax.experimental import pallas as pl
from jax.experimental.pallas import tpu as pltpu
from jax.experimental.pallas import tpu_sc as plsc

T = 16384
E = 256
K = 8
NGROUP = 8
GSIZE = E // NGROUP
NC, NS, L = 2, 16, 16
NW = NC * NS
TPW = T // NW
CHUNK = 128
NCHUNK = TPW // CHUNK
NEG_INF = float("-inf")


def _merge_top16(ka, va, kb, vb):
    kr = lax.rev(kb, (0,))
    vr = lax.rev(vb, (0,))
    ge = ka >= kr
    hk = jnp.where(ge, ka, kr)
    hv = jnp.where(ge, va, vr)
    return plsc.sort_key_val(hk, hv, descending=True)


def _routing_body(lf_hbm, bias_hbm, oi_hbm, ov_hbm,
                  lbuf, bias_v, sc_scr, hk_scr, hid_scr, lo_scr, loid_scr,
                  oi_scr, ov_scr):
    wid = lax.axis_index("s") * NC + lax.axis_index("c")
    iota = lax.iota(jnp.int32, L)
    pltpu.sync_copy(bias_hbm, bias_v)
    tok0 = wid * TPW

    @pl.loop(0, NCHUNK)
    def _chunk(ci):
        base_tok = tok0 + ci * CHUNK
        pltpu.sync_copy(lf_hbm.at[pl.ds(base_tok * E, CHUNK * E)], lbuf)

        @pl.loop(0, CHUNK)
        def _tok(t):
            toff = t * E
            gv = jnp.full((L,), NEG_INF, jnp.float32)
            for g in range(NGROUP):
                xa = lbuf[pl.ds(toff + g * GSIZE, L)]
                xb = lbuf[pl.ds(toff + g * GSIZE + L, L)]
                sa = 1.0 / (1.0 + jnp.exp(-xa))
                sb = 1.0 / (1.0 + jnp.exp(-xb))
                sc_scr[pl.ds(g * GSIZE, L)] = sa
                sc_scr[pl.ds(g * GSIZE + L, L)] = sb
                a = sa + bias_v[pl.ds(g * GSIZE, L)]
                b = sb + bias_v[pl.ds(g * GSIZE + L, L)]
                ge = a >= b
                hi = jnp.where(ge, a, b)
                lo = jnp.where(ge, b, a)
                hi_src = jnp.where(ge, g * GSIZE + iota, g * GSIZE + L + iota)
                lo_src = hi_src ^ L
                lo_scr[pl.ds(g * L, L)] = lo
                loid_scr[pl.ds(g * L, L)] = lo_src
                hk, hid = plsc.sort_key_val(hi, hi_src, descending=True)
                hk_scr[pl.ds(g * L, L)] = hk
                hid_scr[pl.ds(g * L, L)] = hid
                lane0 = hid[0] & (L - 1)
                lo_at = plsc.load_gather(
                    lo_scr, [jnp.full((L,), g * L, jnp.int32) + lane0]
                )
                m2 = jnp.maximum(hk[1], lo_at[0])
                gv = jnp.where(iota == g, hk[0] + m2, gv)
            _, gid = plsc.sort_key_val(gv, iota, descending=True)
            kept = []
            for r in range(4):
                base = gid[r] * L
                hk_r = hk_scr[pl.ds(base, L)]
                hid_r = hid_scr[pl.ds(base, L)]
                lo_r = lo_scr[pl.ds(base, L)]
                loid_r = loid_scr[pl.ds(base, L)]
                lk, lid = plsc.sort_key_val(lo_r, loid_r, descending=True)
                kept.append(_merge_top16(hk_r, hid_r, lk, lid))
            u0 = _merge_top16(*kept[0], *kept[1])
            u1 = _merge_top16(*kept[2], *kept[3])
            fk, fv = _merge_top16(*u0, *u1)
            mask8 = iota < K
            sgath = plsc.load_gather(sc_scr, [fv])
            s8 = jnp.where(mask8, sgath, 0.0)
            denom = jnp.broadcast_to(jnp.sum(s8) + 1e-20, (L,))
            vals = s8 * 2.5 / denom
            keys = jnp.where(mask8, vals, -1.0)
            ok, oi = plsc.sort_key_val(keys, fv, descending=True)
            plsc.store_compressed(ov_scr.at[pl.ds(t * K, L)], ok, mask=mask8)
            plsc.store_compressed(oi_scr.at[pl.ds(t * K, L)], oi, mask=mask8)

        pltpu.sync_copy(ov_scr.at[pl.ds(0, CHUNK * K)],
                        ov_hbm.at[pl.ds(base_tok * K, CHUNK * K)])
        pltpu.sync_copy(oi_scr.at[pl.ds(0, CHUNK * K)],
                        oi_hbm.at[pl.ds(base_tok * K, CHUNK * K)])


_router = functools.partial(
    pl.kernel,
    out_type=(
        jax.ShapeDtypeStruct((T * K,), jnp.int32),
        jax.ShapeDtypeStruct((T * K,), jnp.float32),
    ),
    mesh=plsc.VectorSubcoreMesh(
        core_axis_name="c", subcore_axis_name="s", num_cores=NC, num_subcores=NS
    ),
    compiler_params=pltpu.CompilerParams(needs_layout_passes=False),
    scratch_types=[
        pltpu.VMEM((CHUNK * E,), jnp.float32),
        pltpu.VMEM((E,), jnp.float32),
        pltpu.VMEM((E,), jnp.float32),
        pltpu.VMEM((NGROUP * L,), jnp.float32),
        pltpu.VMEM((NGROUP * L,), jnp.int32),
        pltpu.VMEM((NGROUP * L,), jnp.float32),
        pltpu.VMEM((NGROUP * L,), jnp.int32),
        pltpu.VMEM((CHUNK * K + K,), jnp.int32),
        pltpu.VMEM((CHUNK * K + K,), jnp.float32),
    ],
)(_routing_body)


def kernel(logits, e_score_correction_bias):
    oi, ov = _router(logits.reshape(-1), e_score_correction_bias)
    return oi.reshape(T, K), ov.reshape(T, K)


# speedup vs baseline: 40.0362x; 1.0822x over previous
import functools

import jax
import jax.numpy as jnp
from jax import lax
from jax.experimental import pallas as pl
from jax.experimental.pallas import tpu as pltpu
from jax.experimental.pallas import tpu_sc as plsc

T = 16384
E = 256
K = 8
NGROUP = 8
GSIZE = E // NGROUP
NC, NS, L = 2, 16, 16
NW = NC * NS
TPW = T // NW
CHUNK = 128
NCHUNK = TPW // CHUNK
NEG_INF = float("-inf")


def _merge_top16(ka, va, kb, vb):
    kr = lax.rev(kb, (0,))
    vr = lax.rev(vb, (0,))
    ge = ka >= kr
    hk = jnp.where(ge, ka, kr)
    hv = jnp.where(ge, va, vr)
    return plsc.sort_key_val(hk, hv, descending=True)


def _routing_body(lf_hbm, bias_hbm, oi_hbm, ov_hbm,
                  lbuf, bias_v, sc_scr, hk_scr, hid_scr, lk_scr, lid_scr,
                  oi_scr, ov_scr):
    wid = lax.axis_index("s") * NC + lax.axis_index("c")
    iota = lax.iota(jnp.int32, L)
    pltpu.sync_copy(bias_hbm, bias_v)
    tok0 = wid * TPW

    @pl.loop(0, NCHUNK)
    def _chunk(ci):
        base_tok = tok0 + ci * CHUNK
        pltpu.sync_copy(lf_hbm.at[pl.ds(base_tok * E, CHUNK * E)], lbuf)

        @pl.loop(0, CHUNK)
        def _tok(t):
            toff = t * E
            gv = jnp.full((L,), NEG_INF, jnp.float32)
            for g in range(NGROUP):
                xa = lbuf[pl.ds(toff + g * GSIZE, L)]
                xb = lbuf[pl.ds(toff + g * GSIZE + L, L)]
                sa = 1.0 / (1.0 + jnp.exp(-xa))
                sb = 1.0 / (1.0 + jnp.exp(-xb))
                sc_scr[pl.ds(g * GSIZE, L)] = sa
                sc_scr[pl.ds(g * GSIZE + L, L)] = sb
                a = sa + bias_v[pl.ds(g * GSIZE, L)]
                b = sb + bias_v[pl.ds(g * GSIZE + L, L)]
                ge = a >= b
                hi = jnp.where(ge, a, b)
                lo = jnp.where(ge, b, a)
                hi_src = jnp.where(ge, g * GSIZE + iota, g * GSIZE + L + iota)
                lo_src = hi_src ^ L
                hk, hid = plsc.sort_key_val(hi, hi_src, descending=True)
                hk_scr[pl.ds(g * L, L)] = hk
                hid_scr[pl.ds(g * L, L)] = hid
                lk, lid = plsc.sort_key_val(lo, lo_src, descending=True)
                lk_scr[pl.ds(g * L, L)] = lk
                lid_scr[pl.ds(g * L, L)] = lid
                psm = jnp.max(a + b)
                s = jnp.maximum(hk[0] + hk[1], psm)
                gv = jnp.where(iota == g, s, gv)
            _, gid = plsc.sort_key_val(gv, iota, descending=True)
            kept = []
            for r in range(4):
                base = gid[r] * L
                hk_r = hk_scr[pl.ds(base, L)]
                hid_r = hid_scr[pl.ds(base, L)]
                lk_r = lk_scr[pl.ds(base, L)]
                lid_r = lid_scr[pl.ds(base, L)]
                kept.append(_merge_top16(hk_r, hid_r, lk_r, lid_r))
            u0 = _merge_top16(*kept[0], *kept[1])
            u1 = _merge_top16(*kept[2], *kept[3])
            fk, fv = _merge_top16(*u0, *u1)
            mask8 = iota < K
            sgath = plsc.load_gather(sc_scr, [fv])
            s8 = jnp.where(mask8, sgath, 0.0)
            denom = jnp.broadcast_to(jnp.sum(s8) + 1e-20, (L,))
            vals = s8 * 2.5 / denom
            keys = jnp.where(mask8, vals, -1.0)
            ok, oi = plsc.sort_key_val(keys, fv, descending=True)
            plsc.store_compressed(ov_scr.at[pl.ds(t * K, L)], ok, mask=mask8)
            plsc.store_compressed(oi_scr.at[pl.ds(t * K, L)], oi, mask=mask8)

        pltpu.sync_copy(ov_scr.at[pl.ds(0, CHUNK * K)],
                        ov_hbm.at[pl.ds(base_tok * K, CHUNK * K)])
        pltpu.sync_copy(oi_scr.at[pl.ds(0, CHUNK * K)],
                        oi_hbm.at[pl.ds(base_tok * K, CHUNK * K)])


_router = functools.partial(
    pl.kernel,
    out_type=(
        jax.ShapeDtypeStruct((T * K,), jnp.int32),
        jax.ShapeDtypeStruct((T * K,), jnp.float32),
    ),
    mesh=plsc.VectorSubcoreMesh(
        core_axis_name="c", subcore_axis_name="s", num_cores=NC, num_subcores=NS
    ),
    compiler_params=pltpu.CompilerParams(needs_layout_passes=False),
    scratch_types=[
        pltpu.VMEM((CHUNK * E,), jnp.float32),
        pltpu.VMEM((E,), jnp.float32),
        pltpu.VMEM((E,), jnp.float32),
        pltpu.VMEM((NGROUP * L,), jnp.float32),
        pltpu.VMEM((NGROUP * L,), jnp.int32),
        pltpu.VMEM((NGROUP * L,), jnp.float32),
        pltpu.VMEM((NGROUP * L,), jnp.int32),
        pltpu.VMEM((CHUNK * K + K,), jnp.int32),
        pltpu.VMEM((CHUNK * K + K,), jnp.float32),
    ],
)(_routing_body)


def kernel(logits, e_score_correction_bias):
    oi, ov = _router(logits.reshape(-1), e_score_correction_bias)
    return oi.reshape(T, K), ov.reshape(T, K)
